# baseline (speedup 1.0000x reference)
import functools

import jax
import jax.numpy as jnp
from jax import lax
from jax.experimental import pallas as pl
from jax.experimental.pallas import tpu as pltpu

POOL_SIZE = 100
PROMPT_LENGTH = 10
D_MODEL = 1024
TOP_K = 5
SEQ = 2048
PREFIX = TOP_K * PROMPT_LENGTH


def _body(x_ref, pf_ref, keys_ref, out_ref, idx_ref):
    s = x_ref[0]
    q = jnp.sum(s, axis=0, keepdims=True) * (1.0 / SEQ)
    qn = q / jnp.maximum(jnp.sqrt(jnp.sum(q * q)), 1e-12)

    k = keys_ref[:]
    knorm = jnp.sqrt(jnp.sum(k * k, axis=1, keepdims=True))
    kn = k / jnp.maximum(knorm, 1e-12)

    sim = lax.dot_general(
        qn, kn, (((1,), (1,)), ((), ())), preferred_element_type=jnp.float32
    )

    iota = lax.broadcasted_iota(jnp.int32, (1, POOL_SIZE), 1)
    idxs = []
    cur = sim
    for t in range(TOP_K):
        m = jnp.max(cur)
        it = jnp.min(jnp.where(cur == m, iota, POOL_SIZE))
        idx_ref[0, 0, t] = it
        idxs.append(it)
        cur = jnp.where(iota == it, -jnp.inf, cur)

    r_i = lax.broadcasted_iota(jnp.int32, (PREFIX, POOL_SIZE * PROMPT_LENGTH), 0)
    c_i = lax.broadcasted_iota(jnp.int32, (PREFIX, POOL_SIZE * PROMPT_LENGTH), 1)
    kk = r_i // PROMPT_LENGTH
    within = r_i % PROMPT_LENGTH
    sel_idx = jnp.zeros_like(kk)
    for t, it in enumerate(idxs):
        sel_idx = jnp.where(kk == t, it, sel_idx)
    oh = (c_i == sel_idx * PROMPT_LENGTH + within).astype(jnp.float32)
    sel = lax.dot_general(
        oh, pf_ref[:], (((1,), (0,)), ((), ())), preferred_element_type=jnp.float32
    )

    out_ref[0, 0:PREFIX, :] = sel
    out_ref[0, PREFIX:, :] = s


@functools.partial(jax.jit)
def kernel(x, prompts, keys):
    B = x.shape[0]
    pf = prompts.reshape(POOL_SIZE * PROMPT_LENGTH, D_MODEL)
    out, idx3 = pl.pallas_call(
        _body,
        grid=(B,),
        in_specs=[
            pl.BlockSpec((1, SEQ, D_MODEL), lambda b: (b, 0, 0)),
            pl.BlockSpec((POOL_SIZE * PROMPT_LENGTH, D_MODEL), lambda b: (0, 0)),
            pl.BlockSpec((POOL_SIZE, D_MODEL), lambda b: (0, 0)),
        ],
        out_specs=[
            pl.BlockSpec((1, PREFIX + SEQ, D_MODEL), lambda b: (b, 0, 0)),
            pl.BlockSpec((1, 1, TOP_K), lambda b: (b, 0, 0), memory_space=pltpu.SMEM),
        ],
        out_shape=[
            jax.ShapeDtypeStruct((B, PREFIX + SEQ, D_MODEL), jnp.float32),
            jax.ShapeDtypeStruct((B, 1, TOP_K), jnp.int32),
        ],
    )(x, pf, keys)
    return (out, idx3.reshape(B, TOP_K))
